# baseline (speedup 1.0000x reference)
import jax
import jax.numpy as jnp
from jax import lax
from jax.experimental import pallas as pl
from jax.experimental.pallas import tpu as pltpu
from jax.experimental.pallas import tpu_sc as plsc

N = 10000
E = 160000
IN_DIM = 384
HID = 256
HH = 128
NB = 64
NC = 2
NS = 16
CH = 128
CHUNKS = 80
PER_TILE = CH * CHUNKS
E_PAD = NS * PER_TILE
ACC_ROWS = 10112
RPT = ACC_ROWS // NS
R = 2000
GRID = N // R
HALF = CHUNKS // 2


def _sc_round_body(table, srcs, dsts, zrow,
                   out_sum,
                   acc, src_v, dst_v, rows_a, rows_b, sem_a, sem_b):
    c = lax.axis_index("c")
    t = lax.axis_index("s")
    pltpu.sync_copy(zrow, acc.at[pl.ds(t * RPT, RPT)])
    plsc.subcore_barrier()

    for h in range(2):
        pltpu.sync_copy(srcs.at[c, t, pl.ds(h * HALF, HALF)], src_v)
        pltpu.sync_copy(dsts.at[t, pl.ds(h * HALF, HALF)], dst_v)
        pltpu.async_copy(table.at[src_v.at[0]], rows_a, sem_a)

        def pair(g, carry):
            j = g * 2
            pltpu.async_copy(table.at[src_v.at[j + 1]], rows_b, sem_b)
            pltpu.make_async_copy(table.at[src_v.at[j]], rows_a, sem_a).wait()
            pltpu.sync_copy(rows_a, acc.at[dst_v.at[j]], add=True)

            @pl.when(j + 2 < HALF)
            def _():
                pltpu.async_copy(table.at[src_v.at[j + 2]], rows_a, sem_a)

            pltpu.make_async_copy(table.at[src_v.at[j + 1]], rows_b,
                                  sem_b).wait()
            pltpu.sync_copy(rows_b, acc.at[dst_v.at[j + 1]], add=True)
            return carry

        lax.fori_loop(0, HALF // 2, pair, 0)

    plsc.subcore_barrier()
    pltpu.sync_copy(acc.at[pl.ds(t * RPT, RPT)],
                    out_sum.at[c, pl.ds(t * RPT, RPT)])


def _sc_count_body(dsts2, zcnt, ones_h,
                   out_cnt,
                   cnt, dst_v, ones_v):
    c = lax.axis_index("c")
    t = lax.axis_index("s")
    pltpu.sync_copy(zcnt, cnt.at[pl.ds(t * RPT, RPT)])
    pltpu.sync_copy(dsts2.at[c, t], dst_v)
    pltpu.sync_copy(ones_h, ones_v)
    plsc.subcore_barrier()

    def step(j, carry):
        pltpu.sync_copy(ones_v, cnt.at[dst_v.at[j]], add=True)
        return carry

    lax.fori_loop(0, CHUNKS, step, 0)

    plsc.subcore_barrier()
    pltpu.sync_copy(cnt.at[pl.ds(t * RPT, RPT)],
                    out_cnt.at[c, pl.ds(t * RPT, RPT)])


_sc_cache = {}


def _get_sc_round():
    if "round" not in _sc_cache:
        _sc_cache["round"] = pl.kernel(
            _sc_round_body,
            out_type=jax.ShapeDtypeStruct((NC, ACC_ROWS, HH), jnp.float32),
            mesh=plsc.VectorSubcoreMesh(core_axis_name="c",
                                        subcore_axis_name="s",
                                        num_cores=NC, num_subcores=NS),
            scratch_types=[
                pltpu.VMEM_SHARED((ACC_ROWS, HH), jnp.float32),
                pltpu.VMEM((HALF, CH), jnp.int32),
                pltpu.VMEM((HALF, CH), jnp.int32),
                pltpu.VMEM((CH, HH), jnp.float32),
                pltpu.VMEM((CH, HH), jnp.float32),
                pltpu.SemaphoreType.DMA,
                pltpu.SemaphoreType.DMA,
            ],
        )
    return _sc_cache["round"]


def _get_sc_count():
    if "count" not in _sc_cache:
        _sc_cache["count"] = pl.kernel(
            _sc_count_body,
            out_type=jax.ShapeDtypeStruct((NC, ACC_ROWS, HH), jnp.float32),
            mesh=plsc.VectorSubcoreMesh(core_axis_name="c",
                                        subcore_axis_name="s",
                                        num_cores=NC, num_subcores=NS),
            scratch_types=[
                pltpu.VMEM_SHARED((ACC_ROWS, HH), jnp.float32),
                pltpu.VMEM((CHUNKS, CH), jnp.int32),
                pltpu.VMEM((CH, HH), jnp.float32),
            ],
        )
    return _sc_cache["count"]


def _run_sc_round(table, srcs, dsts, zrow):
    return _get_sc_round()(table, srcs, dsts, zrow)


def _run_sc_count(dsts2, zcnt, ones_h):
    return _get_sc_count()(dsts2, zcnt, ones_h)


def _proj_body(x_ref, w_ref, b_ref, lo_ref, hi_ref):
    h = lax.dot_general(x_ref[...], w_ref[...], (((1,), (1,)), ((), ())),
                        preferred_element_type=jnp.float32) + b_ref[...]
    lo_ref[...] = h[:, :HH]
    hi_ref[...] = h[:, HH:]


def _proj(x, w, b):
    return pl.pallas_call(
        _proj_body,
        grid=(GRID,),
        in_specs=[
            pl.BlockSpec((R, IN_DIM), lambda i: (i, 0)),
            pl.BlockSpec((HID, IN_DIM), lambda i: (0, 0)),
            pl.BlockSpec((1, HID), lambda i: (0, 0)),
        ],
        out_specs=[
            pl.BlockSpec((R, HH), lambda i: (i, 0)),
            pl.BlockSpec((R, HH), lambda i: (i, 0)),
        ],
        out_shape=[jax.ShapeDtypeStruct((N, HH), jnp.float32),
                   jax.ShapeDtypeStruct((N, HH), jnp.float32)],
    )(x, w, b.reshape(1, HID))


def _upd_body(s_lo_ref, s_hi_ref, cnt_ref, h_lo_ref, h_hi_ref,
              wl_ref, wr_ref, bias_ref, o_lo_ref, o_hi_ref):
    c = jnp.maximum(cnt_ref[0, :, 0:1], 1.0)
    mean = jnp.concatenate([s_lo_ref[0], s_hi_ref[0]], axis=1) / c
    h = jnp.concatenate([h_lo_ref[...], h_hi_ref[...]], axis=1)
    o = (lax.dot_general(mean, wl_ref[...], (((1,), (1,)), ((), ())),
                         preferred_element_type=jnp.float32)
         + lax.dot_general(h, wr_ref[...], (((1,), (1,)), ((), ())),
                           preferred_element_type=jnp.float32)
         + bias_ref[...])
    nh = jnp.maximum(o, 0.0) + h
    o_lo_ref[...] = nh[:, :HH]
    o_hi_ref[...] = nh[:, HH:]


def _upd(s2, cnt2, et, h_lo, h_hi, wl, wr, bias):
    return pl.pallas_call(
        _upd_body,
        grid=(GRID,),
        in_specs=[
            pl.BlockSpec((1, R, HH), lambda i: (0, i, 0)),
            pl.BlockSpec((1, R, HH), lambda i: (1, i, 0)),
            pl.BlockSpec((1, R, HH), lambda i, e=et: (e, i, 0)),
            pl.BlockSpec((R, HH), lambda i: (i, 0)),
            pl.BlockSpec((R, HH), lambda i: (i, 0)),
            pl.BlockSpec((HID, HID), lambda i: (0, 0)),
            pl.BlockSpec((HID, HID), lambda i: (0, 0)),
            pl.BlockSpec((1, HID), lambda i: (0, 0)),
        ],
        out_specs=[
            pl.BlockSpec((R, HH), lambda i: (i, 0)),
            pl.BlockSpec((R, HH), lambda i: (i, 0)),
        ],
        out_shape=[jax.ShapeDtypeStruct((N, HH), jnp.float32),
                   jax.ShapeDtypeStruct((N, HH), jnp.float32)],
    )(s2, s2, cnt2, h_lo, h_hi, wl, wr, bias)


def _pool_body(bu_ref, bi_ref, hu_lo_ref, hu_hi_ref, hi_lo_ref, hi_hi_ref,
               wout_ref, bout_ref, out_ref, su, si, cu, ci):
    i = pl.program_id(0)
    ones_r8 = jnp.ones((R, 8), jnp.float32)
    iota = lax.broadcasted_iota(jnp.int32, (NB, R), 0)

    oh_u = (iota == bu_ref[0]).astype(jnp.float32)
    oh_i = (iota == bi_ref[0]).astype(jnp.float32)
    hu = jnp.concatenate([hu_lo_ref[...], hu_hi_ref[...]], axis=1)
    hi = jnp.concatenate([hi_lo_ref[...], hi_hi_ref[...]], axis=1)
    dn = (((1,), (0,)), ((), ()))
    su_p = lax.dot_general(oh_u, hu, dn, preferred_element_type=jnp.float32)
    si_p = lax.dot_general(oh_i, hi, dn, preferred_element_type=jnp.float32)
    cu_p = lax.dot_general(oh_u, ones_r8, dn, preferred_element_type=jnp.float32)
    ci_p = lax.dot_general(oh_i, ones_r8, dn, preferred_element_type=jnp.float32)

    @pl.when(i == 0)
    def _():
        su[...] = su_p
        si[...] = si_p
        cu[...] = cu_p
        ci[...] = ci_p

    @pl.when(i > 0)
    def _():
        su[...] += su_p
        si[...] += si_p
        cu[...] += cu_p
        ci[...] += ci_p

    @pl.when(i == GRID - 1)
    def _():
        mu = su[...] / jnp.maximum(cu[...][:, 0:1], 1.0)
        mi = si[...] / jnp.maximum(ci[...][:, 0:1], 1.0)
        g = (mu + mi) * 0.5
        out_ref[...] = (lax.dot_general(g, wout_ref[...],
                                        (((1,), (1,)), ((), ())),
                                        preferred_element_type=jnp.float32)
                        + bout_ref[...])


def _pool(bu3, bi3, hu_lo, hu_hi, hi_lo, hi_hi, wout, bout):
    return pl.pallas_call(
        _pool_body,
        grid=(GRID,),
        in_specs=[
            pl.BlockSpec((1, 1, R), lambda i: (i, 0, 0)),
            pl.BlockSpec((1, 1, R), lambda i: (i, 0, 0)),
            pl.BlockSpec((R, HH), lambda i: (i, 0)),
            pl.BlockSpec((R, HH), lambda i: (i, 0)),
            pl.BlockSpec((R, HH), lambda i: (i, 0)),
            pl.BlockSpec((R, HH), lambda i: (i, 0)),
            pl.BlockSpec((HID, HID), lambda i: (0, 0)),
            pl.BlockSpec((1, HID), lambda i: (0, 0)),
        ],
        out_specs=pl.BlockSpec((NB, HID), lambda i: (0, 0)),
        out_shape=jax.ShapeDtypeStruct((NB, HID), jnp.float32),
        scratch_shapes=[
            pltpu.VMEM((NB, HID), jnp.float32),
            pltpu.VMEM((NB, HID), jnp.float32),
            pltpu.VMEM((NB, 8), jnp.float32),
            pltpu.VMEM((NB, 8), jnp.float32),
        ],
    )(bu3, bi3, hu_lo, hu_hi, hi_lo, hi_hi, wout, bout.reshape(1, HID))


def _prep_edges(ei):
    src = ei[0].astype(jnp.int32)
    dst = ei[1].astype(jnp.int32)
    pad = E_PAD - E
    src_p = jnp.concatenate([src, jnp.zeros((pad,), jnp.int32)])
    dst_p = jnp.concatenate([dst, jnp.full((pad,), N, jnp.int32)])
    srcs = jnp.stack([src_p, src_p + N]).reshape(NC, NS, CHUNKS, CH)
    dsts = dst_p.reshape(NS, CHUNKS, CH)
    return srcs, dsts


def kernel(x_user, x_item, edge_index_ui, edge_index_iu, batch_user, batch_item,
           Wp_user, bp_user, Wp_item, bp_item,
           Wl_ui_0, bl_ui_0, Wr_ui_0, br_ui_0,
           Wl_iu_0, bl_iu_0, Wr_iu_0, br_iu_0,
           Wl_ui_1, bl_ui_1, Wr_ui_1, br_ui_1,
           Wl_iu_1, bl_iu_1, Wr_iu_1, br_iu_1,
           Wout, bout):
    hu_lo, hu_hi = _proj(x_user, Wp_user, bp_user)
    hi_lo, hi_hi = _proj(x_item, Wp_item, bp_item)

    srcs_ui, dsts_ui = _prep_edges(edge_index_ui)
    srcs_iu, dsts_iu = _prep_edges(edge_index_iu)
    zrow = jnp.zeros((RPT, HH), jnp.float32)
    ones_h = jnp.ones((CH, HH), jnp.float32)
    cnt2 = _run_sc_count(jnp.stack([dsts_ui, dsts_iu]), zrow, ones_h)

    layer_w = [(Wl_ui_0, Wr_ui_0, (bl_ui_0 + br_ui_0).reshape(1, HID),
                Wl_iu_0, Wr_iu_0, (bl_iu_0 + br_iu_0).reshape(1, HID)),
               (Wl_ui_1, Wr_ui_1, (bl_ui_1 + br_ui_1).reshape(1, HID),
                Wl_iu_1, Wr_iu_1, (bl_iu_1 + br_iu_1).reshape(1, HID))]

    for (wl_ui, wr_ui, b_ui, wl_iu, wr_iu, b_iu) in layer_w:
        table_u = jnp.concatenate([hu_lo, hu_hi], axis=0)
        table_i = jnp.concatenate([hi_lo, hi_hi], axis=0)
        s_i = _run_sc_round(table_u, srcs_ui, dsts_ui, zrow)
        s_u = _run_sc_round(table_i, srcs_iu, dsts_iu, zrow)
        hi_lo, hi_hi = _upd(s_i, cnt2, 0, hi_lo, hi_hi, wl_ui, wr_ui, b_ui)
        hu_lo, hu_hi = _upd(s_u, cnt2, 1, hu_lo, hu_hi, wl_iu, wr_iu, b_iu)

    bu3 = batch_user.astype(jnp.int32).reshape(GRID, 1, R)
    bi3 = batch_item.astype(jnp.int32).reshape(GRID, 1, R)
    return _pool(bu3, bi3, hu_lo, hu_hi, hi_lo, hi_hi, Wout, bout)


# speedup vs baseline: 1.0090x; 1.0090x over previous
import jax
import jax.numpy as jnp
from jax import lax
from jax.experimental import pallas as pl
from jax.experimental.pallas import tpu as pltpu
from jax.experimental.pallas import tpu_sc as plsc

N = 10000
E = 160000
IN_DIM = 384
HID = 256
HH = 128
NB = 64
NC = 2
NS = 16
CH = 128
CHUNKS = 80
PER_TILE = CH * CHUNKS
E_PAD = NS * PER_TILE
ACC_ROWS = 10112
RPT = ACC_ROWS // NS
R = 2000
GRID = N // R
HALF = CHUNKS // 2


def _sc_round_body(tab_lo, tab_hi, srcs, dsts, zrow,
                   out_sum,
                   acc, src_v, dst_v, rows_a, rows_b, sem_a, sem_b):
    c = lax.axis_index("c")
    t = lax.axis_index("s")
    pltpu.sync_copy(zrow, acc.at[pl.ds(t * RPT, RPT)])
    plsc.subcore_barrier()

    def run_half(h, table):
        pltpu.sync_copy(srcs.at[t, pl.ds(h * HALF, HALF)], src_v)
        pltpu.sync_copy(dsts.at[t, pl.ds(h * HALF, HALF)], dst_v)
        pltpu.async_copy(table.at[src_v.at[0]], rows_a, sem_a)

        def pair(g, carry):
            j = g * 2
            pltpu.async_copy(table.at[src_v.at[j + 1]], rows_b, sem_b)
            pltpu.make_async_copy(table.at[src_v.at[j]], rows_a, sem_a).wait()
            pltpu.sync_copy(rows_a, acc.at[dst_v.at[j]], add=True)

            @pl.when(j + 2 < HALF)
            def _():
                pltpu.async_copy(table.at[src_v.at[j + 2]], rows_a, sem_a)

            pltpu.make_async_copy(table.at[src_v.at[j + 1]], rows_b,
                                  sem_b).wait()
            pltpu.sync_copy(rows_b, acc.at[dst_v.at[j + 1]], add=True)
            return carry

        lax.fori_loop(0, HALF // 2, pair, 0)

    for h in range(2):
        @pl.when(c == 0)
        def _():
            run_half(h, tab_lo)

        @pl.when(c == 1)
        def _():
            run_half(h, tab_hi)

    plsc.subcore_barrier()
    pltpu.sync_copy(acc.at[pl.ds(t * RPT, RPT)],
                    out_sum.at[c, pl.ds(t * RPT, RPT)])


def _sc_count_body(dsts2, zcnt, ones_h,
                   out_cnt,
                   cnt, dst_v, ones_v):
    c = lax.axis_index("c")
    t = lax.axis_index("s")
    pltpu.sync_copy(zcnt, cnt.at[pl.ds(t * RPT, RPT)])
    pltpu.sync_copy(dsts2.at[c, t], dst_v)
    pltpu.sync_copy(ones_h, ones_v)
    plsc.subcore_barrier()

    def step(j, carry):
        pltpu.sync_copy(ones_v, cnt.at[dst_v.at[j]], add=True)
        return carry

    lax.fori_loop(0, CHUNKS, step, 0)

    plsc.subcore_barrier()
    pltpu.sync_copy(cnt.at[pl.ds(t * RPT, RPT)],
                    out_cnt.at[c, pl.ds(t * RPT, RPT)])


_sc_cache = {}


def _get_sc_round():
    if "round" not in _sc_cache:
        _sc_cache["round"] = pl.kernel(
            _sc_round_body,
            out_type=jax.ShapeDtypeStruct((NC, ACC_ROWS, HH), jnp.float32),
            mesh=plsc.VectorSubcoreMesh(core_axis_name="c",
                                        subcore_axis_name="s",
                                        num_cores=NC, num_subcores=NS),
            scratch_types=[
                pltpu.VMEM_SHARED((ACC_ROWS, HH), jnp.float32),
                pltpu.VMEM((HALF, CH), jnp.int32),
                pltpu.VMEM((HALF, CH), jnp.int32),
                pltpu.VMEM((CH, HH), jnp.float32),
                pltpu.VMEM((CH, HH), jnp.float32),
                pltpu.SemaphoreType.DMA,
                pltpu.SemaphoreType.DMA,
            ],
        )
    return _sc_cache["round"]


def _get_sc_count():
    if "count" not in _sc_cache:
        _sc_cache["count"] = pl.kernel(
            _sc_count_body,
            out_type=jax.ShapeDtypeStruct((NC, ACC_ROWS, HH), jnp.float32),
            mesh=plsc.VectorSubcoreMesh(core_axis_name="c",
                                        subcore_axis_name="s",
                                        num_cores=NC, num_subcores=NS),
            scratch_types=[
                pltpu.VMEM_SHARED((ACC_ROWS, HH), jnp.float32),
                pltpu.VMEM((CHUNKS, CH), jnp.int32),
                pltpu.VMEM((CH, HH), jnp.float32),
            ],
        )
    return _sc_cache["count"]


def _run_sc_round(tab_lo, tab_hi, srcs, dsts, zrow):
    return _get_sc_round()(tab_lo, tab_hi, srcs, dsts, zrow)


def _run_sc_count(dsts2, zcnt, ones_h):
    return _get_sc_count()(dsts2, zcnt, ones_h)


def _proj_body(x_ref, w_ref, b_ref, lo_ref, hi_ref):
    h = lax.dot_general(x_ref[...], w_ref[...], (((1,), (1,)), ((), ())),
                        preferred_element_type=jnp.float32) + b_ref[...]
    lo_ref[...] = h[:, :HH]
    hi_ref[...] = h[:, HH:]


def _proj(x, w, b):
    return pl.pallas_call(
        _proj_body,
        grid=(GRID,),
        in_specs=[
            pl.BlockSpec((R, IN_DIM), lambda i: (i, 0)),
            pl.BlockSpec((HID, IN_DIM), lambda i: (0, 0)),
            pl.BlockSpec((1, HID), lambda i: (0, 0)),
        ],
        out_specs=[
            pl.BlockSpec((R, HH), lambda i: (i, 0)),
            pl.BlockSpec((R, HH), lambda i: (i, 0)),
        ],
        out_shape=[jax.ShapeDtypeStruct((N, HH), jnp.float32),
                   jax.ShapeDtypeStruct((N, HH), jnp.float32)],
    )(x, w, b.reshape(1, HID))


def _upd_body(s_lo_ref, s_hi_ref, cnt_ref, h_lo_ref, h_hi_ref,
              wl_ref, wr_ref, bias_ref, o_lo_ref, o_hi_ref):
    c = jnp.maximum(cnt_ref[0, :, 0:1], 1.0)
    mean = jnp.concatenate([s_lo_ref[0], s_hi_ref[0]], axis=1) / c
    h = jnp.concatenate([h_lo_ref[...], h_hi_ref[...]], axis=1)
    o = (lax.dot_general(mean, wl_ref[...], (((1,), (1,)), ((), ())),
                         preferred_element_type=jnp.float32)
         + lax.dot_general(h, wr_ref[...], (((1,), (1,)), ((), ())),
                           preferred_element_type=jnp.float32)
         + bias_ref[...])
    nh = jnp.maximum(o, 0.0) + h
    o_lo_ref[...] = nh[:, :HH]
    o_hi_ref[...] = nh[:, HH:]


def _upd(s2, cnt2, et, h_lo, h_hi, wl, wr, bias):
    return pl.pallas_call(
        _upd_body,
        grid=(GRID,),
        in_specs=[
            pl.BlockSpec((1, R, HH), lambda i: (0, i, 0)),
            pl.BlockSpec((1, R, HH), lambda i: (1, i, 0)),
            pl.BlockSpec((1, R, HH), lambda i, e=et: (e, i, 0)),
            pl.BlockSpec((R, HH), lambda i: (i, 0)),
            pl.BlockSpec((R, HH), lambda i: (i, 0)),
            pl.BlockSpec((HID, HID), lambda i: (0, 0)),
            pl.BlockSpec((HID, HID), lambda i: (0, 0)),
            pl.BlockSpec((1, HID), lambda i: (0, 0)),
        ],
        out_specs=[
            pl.BlockSpec((R, HH), lambda i: (i, 0)),
            pl.BlockSpec((R, HH), lambda i: (i, 0)),
        ],
        out_shape=[jax.ShapeDtypeStruct((N, HH), jnp.float32),
                   jax.ShapeDtypeStruct((N, HH), jnp.float32)],
    )(s2, s2, cnt2, h_lo, h_hi, wl, wr, bias)


def _pool_body(bu_ref, bi_ref, hu_lo_ref, hu_hi_ref, hi_lo_ref, hi_hi_ref,
               wout_ref, bout_ref, out_ref, su, si, cu, ci):
    i = pl.program_id(0)
    ones_r8 = jnp.ones((R, 8), jnp.float32)
    iota = lax.broadcasted_iota(jnp.int32, (NB, R), 0)

    oh_u = (iota == bu_ref[0]).astype(jnp.float32)
    oh_i = (iota == bi_ref[0]).astype(jnp.float32)
    hu = jnp.concatenate([hu_lo_ref[...], hu_hi_ref[...]], axis=1)
    hi = jnp.concatenate([hi_lo_ref[...], hi_hi_ref[...]], axis=1)
    dn = (((1,), (0,)), ((), ()))
    su_p = lax.dot_general(oh_u, hu, dn, preferred_element_type=jnp.float32)
    si_p = lax.dot_general(oh_i, hi, dn, preferred_element_type=jnp.float32)
    cu_p = lax.dot_general(oh_u, ones_r8, dn, preferred_element_type=jnp.float32)
    ci_p = lax.dot_general(oh_i, ones_r8, dn, preferred_element_type=jnp.float32)

    @pl.when(i == 0)
    def _():
        su[...] = su_p
        si[...] = si_p
        cu[...] = cu_p
        ci[...] = ci_p

    @pl.when(i > 0)
    def _():
        su[...] += su_p
        si[...] += si_p
        cu[...] += cu_p
        ci[...] += ci_p

    @pl.when(i == GRID - 1)
    def _():
        mu = su[...] / jnp.maximum(cu[...][:, 0:1], 1.0)
        mi = si[...] / jnp.maximum(ci[...][:, 0:1], 1.0)
        g = (mu + mi) * 0.5
        out_ref[...] = (lax.dot_general(g, wout_ref[...],
                                        (((1,), (1,)), ((), ())),
                                        preferred_element_type=jnp.float32)
                        + bout_ref[...])


def _pool(bu3, bi3, hu_lo, hu_hi, hi_lo, hi_hi, wout, bout):
    return pl.pallas_call(
        _pool_body,
        grid=(GRID,),
        in_specs=[
            pl.BlockSpec((1, 1, R), lambda i: (i, 0, 0)),
            pl.BlockSpec((1, 1, R), lambda i: (i, 0, 0)),
            pl.BlockSpec((R, HH), lambda i: (i, 0)),
            pl.BlockSpec((R, HH), lambda i: (i, 0)),
            pl.BlockSpec((R, HH), lambda i: (i, 0)),
            pl.BlockSpec((R, HH), lambda i: (i, 0)),
            pl.BlockSpec((HID, HID), lambda i: (0, 0)),
            pl.BlockSpec((1, HID), lambda i: (0, 0)),
        ],
        out_specs=pl.BlockSpec((NB, HID), lambda i: (0, 0)),
        out_shape=jax.ShapeDtypeStruct((NB, HID), jnp.float32),
        scratch_shapes=[
            pltpu.VMEM((NB, HID), jnp.float32),
            pltpu.VMEM((NB, HID), jnp.float32),
            pltpu.VMEM((NB, 8), jnp.float32),
            pltpu.VMEM((NB, 8), jnp.float32),
        ],
    )(bu3, bi3, hu_lo, hu_hi, hi_lo, hi_hi, wout, bout.reshape(1, HID))


def _prep_edges(ei):
    src = ei[0].astype(jnp.int32)
    dst = ei[1].astype(jnp.int32)
    pad = E_PAD - E
    src_p = jnp.concatenate([src, jnp.zeros((pad,), jnp.int32)])
    dst_p = jnp.concatenate([dst, jnp.full((pad,), N, jnp.int32)])
    srcs = src_p.reshape(NS, CHUNKS, CH)
    dsts = dst_p.reshape(NS, CHUNKS, CH)
    return srcs, dsts


def kernel(x_user, x_item, edge_index_ui, edge_index_iu, batch_user, batch_item,
           Wp_user, bp_user, Wp_item, bp_item,
           Wl_ui_0, bl_ui_0, Wr_ui_0, br_ui_0,
           Wl_iu_0, bl_iu_0, Wr_iu_0, br_iu_0,
           Wl_ui_1, bl_ui_1, Wr_ui_1, br_ui_1,
           Wl_iu_1, bl_iu_1, Wr_iu_1, br_iu_1,
           Wout, bout):
    hu_lo, hu_hi = _proj(x_user, Wp_user, bp_user)
    hi_lo, hi_hi = _proj(x_item, Wp_item, bp_item)

    srcs_ui, dsts_ui = _prep_edges(edge_index_ui)
    srcs_iu, dsts_iu = _prep_edges(edge_index_iu)
    zrow = jnp.zeros((RPT, HH), jnp.float32)
    ones_h = jnp.ones((CH, HH), jnp.float32)
    cnt2 = _run_sc_count(jnp.stack([dsts_ui, dsts_iu]), zrow, ones_h)

    layer_w = [(Wl_ui_0, Wr_ui_0, (bl_ui_0 + br_ui_0).reshape(1, HID),
                Wl_iu_0, Wr_iu_0, (bl_iu_0 + br_iu_0).reshape(1, HID)),
               (Wl_ui_1, Wr_ui_1, (bl_ui_1 + br_ui_1).reshape(1, HID),
                Wl_iu_1, Wr_iu_1, (bl_iu_1 + br_iu_1).reshape(1, HID))]

    for (wl_ui, wr_ui, b_ui, wl_iu, wr_iu, b_iu) in layer_w:
        s_i = _run_sc_round(hu_lo, hu_hi, srcs_ui, dsts_ui, zrow)
        s_u = _run_sc_round(hi_lo, hi_hi, srcs_iu, dsts_iu, zrow)
        hi_lo, hi_hi = _upd(s_i, cnt2, 0, hi_lo, hi_hi, wl_ui, wr_ui, b_ui)
        hu_lo, hu_hi = _upd(s_u, cnt2, 1, hu_lo, hu_hi, wl_iu, wr_iu, b_iu)

    bu3 = batch_user.astype(jnp.int32).reshape(GRID, 1, R)
    bi3 = batch_item.astype(jnp.int32).reshape(GRID, 1, R)
    return _pool(bu3, bi3, hu_lo, hu_hi, hi_lo, hi_hi, Wout, bout)


# speedup vs baseline: 1.0248x; 1.0157x over previous
import jax
import jax.numpy as jnp
from jax import lax
from jax.experimental import pallas as pl
from jax.experimental.pallas import tpu as pltpu
from jax.experimental.pallas import tpu_sc as plsc

N = 10000
E = 160000
IN_DIM = 384
HID = 256
HH = 128
NB = 64
NC = 2
NS = 16
CH = 128
CHUNKS = 80
PER_TILE = CH * CHUNKS
E_PAD = NS * PER_TILE
ACC_ROWS = 10112
RPT = ACC_ROWS // NS
R = 2000
GRID = N // R
HALF = CHUNKS // 2


def _sc_round_body(tab_lo, tab_hi, srcs, dsts, zrow,
                   out_sum,
                   acc, src_v, dst_v, rows_a, rows_b, sem_a, sem_b):
    c = lax.axis_index("c")
    t = lax.axis_index("s")
    pltpu.sync_copy(zrow, acc.at[pl.ds(t * RPT, RPT)])
    plsc.subcore_barrier()

    def run_half(h, table):
        pltpu.sync_copy(srcs.at[t, pl.ds(h * HALF, HALF)], src_v)
        pltpu.sync_copy(dsts.at[t, pl.ds(h * HALF, HALF)], dst_v)
        pltpu.async_copy(table.at[src_v.at[0]], rows_a, sem_a)

        def pair(g, carry):
            j = g * 2
            pltpu.async_copy(table.at[src_v.at[j + 1]], rows_b, sem_b)
            pltpu.make_async_copy(table.at[src_v.at[j]], rows_a, sem_a).wait()
            pltpu.sync_copy(rows_a, acc.at[dst_v.at[j]], add=True)

            @pl.when(j + 2 < HALF)
            def _():
                pltpu.async_copy(table.at[src_v.at[j + 2]], rows_a, sem_a)

            pltpu.make_async_copy(table.at[src_v.at[j + 1]], rows_b,
                                  sem_b).wait()
            pltpu.sync_copy(rows_b, acc.at[dst_v.at[j + 1]], add=True)
            return carry

        lax.fori_loop(0, HALF // 2, pair, 0)

    for h in range(2):
        @pl.when(c == 0)
        def _():
            run_half(h, tab_lo)

        @pl.when(c == 1)
        def _():
            run_half(h, tab_hi)

    plsc.subcore_barrier()
    pltpu.sync_copy(acc.at[pl.ds(t * RPT, RPT)],
                    out_sum.at[c, pl.ds(t * RPT, RPT)])


def _sc_count_body(dsts, zcnt, ones_h,
                   out_cnt,
                   cnt, dst_v, ones_v):
    c = lax.axis_index("c")
    t = lax.axis_index("s")

    @pl.when(c == 0)
    def _():
        pltpu.sync_copy(zcnt, cnt.at[pl.ds(t * RPT, RPT)])
        pltpu.sync_copy(dsts.at[t], dst_v)
        pltpu.sync_copy(ones_h, ones_v)

    plsc.subcore_barrier()

    @pl.when(c == 0)
    def _():
        def step(j, carry):
            pltpu.sync_copy(ones_v, cnt.at[dst_v.at[j]], add=True)
            return carry

        lax.fori_loop(0, CHUNKS, step, 0)

    plsc.subcore_barrier()

    @pl.when(c == 0)
    def _():
        pltpu.sync_copy(cnt.at[pl.ds(t * RPT, RPT)],
                        out_cnt.at[pl.ds(t * RPT, RPT)])


_sc_cache = {}


def _get_sc_round():
    if "round" not in _sc_cache:
        _sc_cache["round"] = pl.kernel(
            _sc_round_body,
            out_type=jax.ShapeDtypeStruct((NC, ACC_ROWS, HH), jnp.float32),
            mesh=plsc.VectorSubcoreMesh(core_axis_name="c",
                                        subcore_axis_name="s",
                                        num_cores=NC, num_subcores=NS),
            scratch_types=[
                pltpu.VMEM_SHARED((ACC_ROWS, HH), jnp.float32),
                pltpu.VMEM((HALF, CH), jnp.int32),
                pltpu.VMEM((HALF, CH), jnp.int32),
                pltpu.VMEM((CH, HH), jnp.float32),
                pltpu.VMEM((CH, HH), jnp.float32),
                pltpu.SemaphoreType.DMA,
                pltpu.SemaphoreType.DMA,
            ],
        )
    return _sc_cache["round"]


def _get_sc_count():
    if "count" not in _sc_cache:
        _sc_cache["count"] = pl.kernel(
            _sc_count_body,
            out_type=jax.ShapeDtypeStruct((ACC_ROWS, HH), jnp.float32),
            mesh=plsc.VectorSubcoreMesh(core_axis_name="c",
                                        subcore_axis_name="s",
                                        num_cores=NC, num_subcores=NS),
            scratch_types=[
                pltpu.VMEM_SHARED((ACC_ROWS, HH), jnp.float32),
                pltpu.VMEM((CHUNKS, CH), jnp.int32),
                pltpu.VMEM((CH, HH), jnp.float32),
            ],
        )
    return _sc_cache["count"]


def _run_sc_round(tab_lo, tab_hi, srcs, dsts, zrow):
    return _get_sc_round()(tab_lo, tab_hi, srcs, dsts, zrow)


def _run_sc_count(dsts, zcnt, ones_h):
    return _get_sc_count()(dsts, zcnt, ones_h)


def _proj_body(x_ref, w_ref, b_ref, lo_ref, hi_ref):
    h = lax.dot_general(x_ref[...], w_ref[...], (((1,), (1,)), ((), ())),
                        preferred_element_type=jnp.float32) + b_ref[...]
    lo_ref[...] = h[:, :HH]
    hi_ref[...] = h[:, HH:]


def _proj(x, w, b):
    return pl.pallas_call(
        _proj_body,
        grid=(GRID,),
        in_specs=[
            pl.BlockSpec((R, IN_DIM), lambda i: (i, 0)),
            pl.BlockSpec((HID, IN_DIM), lambda i: (0, 0)),
            pl.BlockSpec((1, HID), lambda i: (0, 0)),
        ],
        out_specs=[
            pl.BlockSpec((R, HH), lambda i: (i, 0)),
            pl.BlockSpec((R, HH), lambda i: (i, 0)),
        ],
        out_shape=[jax.ShapeDtypeStruct((N, HH), jnp.float32),
                   jax.ShapeDtypeStruct((N, HH), jnp.float32)],
    )(x, w, b.reshape(1, HID))


def _upd_body(s_lo_ref, s_hi_ref, cnt_ref, h_lo_ref, h_hi_ref,
              wl_ref, wr_ref, bias_ref, o_lo_ref, o_hi_ref):
    c = jnp.maximum(cnt_ref[:, 0:1], 1.0)
    mean = jnp.concatenate([s_lo_ref[0], s_hi_ref[0]], axis=1) / c
    h = jnp.concatenate([h_lo_ref[...], h_hi_ref[...]], axis=1)
    o = (lax.dot_general(mean, wl_ref[...], (((1,), (1,)), ((), ())),
                         preferred_element_type=jnp.float32)
         + lax.dot_general(h, wr_ref[...], (((1,), (1,)), ((), ())),
                           preferred_element_type=jnp.float32)
         + bias_ref[...])
    nh = jnp.maximum(o, 0.0) + h
    o_lo_ref[...] = nh[:, :HH]
    o_hi_ref[...] = nh[:, HH:]


def _upd(s2, cnt, h_lo, h_hi, wl, wr, bias):
    return pl.pallas_call(
        _upd_body,
        grid=(GRID,),
        in_specs=[
            pl.BlockSpec((1, R, HH), lambda i: (0, i, 0)),
            pl.BlockSpec((1, R, HH), lambda i: (1, i, 0)),
            pl.BlockSpec((R, HH), lambda i: (i, 0)),
            pl.BlockSpec((R, HH), lambda i: (i, 0)),
            pl.BlockSpec((R, HH), lambda i: (i, 0)),
            pl.BlockSpec((HID, HID), lambda i: (0, 0)),
            pl.BlockSpec((HID, HID), lambda i: (0, 0)),
            pl.BlockSpec((1, HID), lambda i: (0, 0)),
        ],
        out_specs=[
            pl.BlockSpec((R, HH), lambda i: (i, 0)),
            pl.BlockSpec((R, HH), lambda i: (i, 0)),
        ],
        out_shape=[jax.ShapeDtypeStruct((N, HH), jnp.float32),
                   jax.ShapeDtypeStruct((N, HH), jnp.float32)],
    )(s2, s2, cnt, h_lo, h_hi, wl, wr, bias)


def _pool_body(bu_ref, bi_ref, hu_lo_ref, hu_hi_ref, hi_lo_ref, hi_hi_ref,
               wout_ref, bout_ref, out_ref, su, si, cu, ci):
    i = pl.program_id(0)
    ones_r8 = jnp.ones((R, 8), jnp.float32)
    iota = lax.broadcasted_iota(jnp.int32, (NB, R), 0)

    oh_u = (iota == bu_ref[0]).astype(jnp.float32)
    oh_i = (iota == bi_ref[0]).astype(jnp.float32)
    hu = jnp.concatenate([hu_lo_ref[...], hu_hi_ref[...]], axis=1)
    hi = jnp.concatenate([hi_lo_ref[...], hi_hi_ref[...]], axis=1)
    dn = (((1,), (0,)), ((), ()))
    su_p = lax.dot_general(oh_u, hu, dn, preferred_element_type=jnp.float32)
    si_p = lax.dot_general(oh_i, hi, dn, preferred_element_type=jnp.float32)
    cu_p = lax.dot_general(oh_u, ones_r8, dn, preferred_element_type=jnp.float32)
    ci_p = lax.dot_general(oh_i, ones_r8, dn, preferred_element_type=jnp.float32)

    @pl.when(i == 0)
    def _():
        su[...] = su_p
        si[...] = si_p
        cu[...] = cu_p
        ci[...] = ci_p

    @pl.when(i > 0)
    def _():
        su[...] += su_p
        si[...] += si_p
        cu[...] += cu_p
        ci[...] += ci_p

    @pl.when(i == GRID - 1)
    def _():
        mu = su[...] / jnp.maximum(cu[...][:, 0:1], 1.0)
        mi = si[...] / jnp.maximum(ci[...][:, 0:1], 1.0)
        g = (mu + mi) * 0.5
        out_ref[...] = (lax.dot_general(g, wout_ref[...],
                                        (((1,), (1,)), ((), ())),
                                        preferred_element_type=jnp.float32)
                        + bout_ref[...])


def _pool(bu3, bi3, hu_lo, hu_hi, hi_lo, hi_hi, wout, bout):
    return pl.pallas_call(
        _pool_body,
        grid=(GRID,),
        in_specs=[
            pl.BlockSpec((1, 1, R), lambda i: (i, 0, 0)),
            pl.BlockSpec((1, 1, R), lambda i: (i, 0, 0)),
            pl.BlockSpec((R, HH), lambda i: (i, 0)),
            pl.BlockSpec((R, HH), lambda i: (i, 0)),
            pl.BlockSpec((R, HH), lambda i: (i, 0)),
            pl.BlockSpec((R, HH), lambda i: (i, 0)),
            pl.BlockSpec((HID, HID), lambda i: (0, 0)),
            pl.BlockSpec((1, HID), lambda i: (0, 0)),
        ],
        out_specs=pl.BlockSpec((NB, HID), lambda i: (0, 0)),
        out_shape=jax.ShapeDtypeStruct((NB, HID), jnp.float32),
        scratch_shapes=[
            pltpu.VMEM((NB, HID), jnp.float32),
            pltpu.VMEM((NB, HID), jnp.float32),
            pltpu.VMEM((NB, 8), jnp.float32),
            pltpu.VMEM((NB, 8), jnp.float32),
        ],
    )(bu3, bi3, hu_lo, hu_hi, hi_lo, hi_hi, wout, bout.reshape(1, HID))


def _prep_edges(ei):
    src = ei[0].astype(jnp.int32)
    dst = ei[1].astype(jnp.int32)
    pad = E_PAD - E
    src_p = jnp.concatenate([src, jnp.zeros((pad,), jnp.int32)])
    dst_p = jnp.concatenate([dst, jnp.full((pad,), N, jnp.int32)])
    srcs = src_p.reshape(NS, CHUNKS, CH)
    dsts = dst_p.reshape(NS, CHUNKS, CH)
    return srcs, dsts


def kernel(x_user, x_item, edge_index_ui, edge_index_iu, batch_user, batch_item,
           Wp_user, bp_user, Wp_item, bp_item,
           Wl_ui_0, bl_ui_0, Wr_ui_0, br_ui_0,
           Wl_iu_0, bl_iu_0, Wr_iu_0, br_iu_0,
           Wl_ui_1, bl_ui_1, Wr_ui_1, br_ui_1,
           Wl_iu_1, bl_iu_1, Wr_iu_1, br_iu_1,
           Wout, bout):
    hu_lo, hu_hi = _proj(x_user, Wp_user, bp_user)
    hi_lo, hi_hi = _proj(x_item, Wp_item, bp_item)

    srcs_ui, dsts_ui = _prep_edges(edge_index_ui)
    srcs_iu, dsts_iu = _prep_edges(edge_index_iu)
    zrow = jnp.zeros((RPT, HH), jnp.float32)
    ones_h = jnp.ones((CH, HH), jnp.float32)
    c_i = _run_sc_count(dsts_ui, zrow, ones_h)
    c_u = _run_sc_count(dsts_iu, zrow, ones_h)

    layer_w = [(Wl_ui_0, Wr_ui_0, (bl_ui_0 + br_ui_0).reshape(1, HID),
                Wl_iu_0, Wr_iu_0, (bl_iu_0 + br_iu_0).reshape(1, HID)),
               (Wl_ui_1, Wr_ui_1, (bl_ui_1 + br_ui_1).reshape(1, HID),
                Wl_iu_1, Wr_iu_1, (bl_iu_1 + br_iu_1).reshape(1, HID))]

    for (wl_ui, wr_ui, b_ui, wl_iu, wr_iu, b_iu) in layer_w:
        s_i = _run_sc_round(hu_lo, hu_hi, srcs_ui, dsts_ui, zrow)
        s_u = _run_sc_round(hi_lo, hi_hi, srcs_iu, dsts_iu, zrow)
        hi_lo, hi_hi = _upd(s_i, c_i[:N], hi_lo, hi_hi, wl_ui, wr_ui, b_ui)
        hu_lo, hu_hi = _upd(s_u, c_u[:N], hu_lo, hu_hi, wl_iu, wr_iu, b_iu)

    bu3 = batch_user.astype(jnp.int32).reshape(GRID, 1, R)
    bi3 = batch_item.astype(jnp.int32).reshape(GRID, 1, R)
    return _pool(bu3, bi3, hu_lo, hu_hi, hi_lo, hi_hi, Wout, bout)


# speedup vs baseline: 1.0485x; 1.0231x over previous
import jax
import jax.numpy as jnp
from jax import lax
from jax.experimental import pallas as pl
from jax.experimental.pallas import tpu as pltpu
from jax.experimental.pallas import tpu_sc as plsc

N = 10000
E = 160000
IN_DIM = 384
HID = 256
HH = 128
NB = 64
NC = 2
NS = 16
CH = 128
CHUNKS = 80
PER_TILE = CH * CHUNKS
E_PAD = NS * PER_TILE
ACC_ROWS = 10112
RPT = ACC_ROWS // NS
R = 2000
GRID = N // R
HALF = CHUNKS // 2


def _sc_round_body(table, srcs, dsts, zrow,
                   out_sum,
                   acc, src_v, dst_v, rows_a, rows_b, sem_a, sem_b):
    c = lax.axis_index("c")
    t = lax.axis_index("s")
    pltpu.sync_copy(zrow, acc.at[pl.ds(t * RPT, RPT)])
    plsc.subcore_barrier()

    for h in range(2):
        pltpu.sync_copy(srcs.at[c, t, pl.ds(h * HALF, HALF)], src_v)
        pltpu.sync_copy(dsts.at[t, pl.ds(h * HALF, HALF)], dst_v)
        pltpu.async_copy(table.at[src_v.at[0]], rows_a, sem_a)

        def pair(g, carry):
            j = g * 2
            pltpu.async_copy(table.at[src_v.at[j + 1]], rows_b, sem_b)
            pltpu.make_async_copy(table.at[src_v.at[j]], rows_a, sem_a).wait()
            pltpu.sync_copy(rows_a, acc.at[dst_v.at[j]], add=True)

            @pl.when(j + 2 < HALF)
            def _():
                pltpu.async_copy(table.at[src_v.at[j + 2]], rows_a, sem_a)

            pltpu.make_async_copy(table.at[src_v.at[j + 1]], rows_b,
                                  sem_b).wait()
            pltpu.sync_copy(rows_b, acc.at[dst_v.at[j + 1]], add=True)
            return carry

        lax.fori_loop(0, HALF // 2, pair, 0)

    plsc.subcore_barrier()
    pltpu.sync_copy(acc.at[pl.ds(t * RPT, RPT)],
                    out_sum.at[c, pl.ds(t * RPT, RPT)])


def _sc_count_body(dsts, zcnt, ones_h,
                   out_cnt,
                   cnt, dst_v, ones_v):
    c = lax.axis_index("c")
    t = lax.axis_index("s")

    @pl.when(c == 0)
    def _():
        pltpu.sync_copy(zcnt, cnt.at[pl.ds(t * RPT, RPT)])
        pltpu.sync_copy(dsts.at[t], dst_v)
        pltpu.sync_copy(ones_h, ones_v)

    plsc.subcore_barrier()

    @pl.when(c == 0)
    def _():
        def step(j, carry):
            pltpu.sync_copy(ones_v, cnt.at[dst_v.at[j]], add=True)
            return carry

        lax.fori_loop(0, CHUNKS, step, 0)

    plsc.subcore_barrier()

    @pl.when(c == 0)
    def _():
        pltpu.sync_copy(cnt.at[pl.ds(t * RPT, RPT)],
                        out_cnt.at[pl.ds(t * RPT, RPT)])


_sc_cache = {}


def _get_sc_round():
    if "round" not in _sc_cache:
        _sc_cache["round"] = pl.kernel(
            _sc_round_body,
            out_type=jax.ShapeDtypeStruct((NC, ACC_ROWS, HH), jnp.float32),
            mesh=plsc.VectorSubcoreMesh(core_axis_name="c",
                                        subcore_axis_name="s",
                                        num_cores=NC, num_subcores=NS),
            scratch_types=[
                pltpu.VMEM_SHARED((ACC_ROWS, HH), jnp.float32),
                pltpu.VMEM((HALF, CH), jnp.int32),
                pltpu.VMEM((HALF, CH), jnp.int32),
                pltpu.VMEM((CH, HH), jnp.float32),
                pltpu.VMEM((CH, HH), jnp.float32),
                pltpu.SemaphoreType.DMA,
                pltpu.SemaphoreType.DMA,
            ],
        )
    return _sc_cache["round"]


def _get_sc_count():
    if "count" not in _sc_cache:
        _sc_cache["count"] = pl.kernel(
            _sc_count_body,
            out_type=jax.ShapeDtypeStruct((ACC_ROWS, HH), jnp.float32),
            mesh=plsc.VectorSubcoreMesh(core_axis_name="c",
                                        subcore_axis_name="s",
                                        num_cores=NC, num_subcores=NS),
            scratch_types=[
                pltpu.VMEM_SHARED((ACC_ROWS, HH), jnp.float32),
                pltpu.VMEM((CHUNKS, CH), jnp.int32),
                pltpu.VMEM((CH, HH), jnp.float32),
            ],
        )
    return _sc_cache["count"]


def _run_sc_round(table, srcs, dsts, zrow):
    return _get_sc_round()(table, srcs, dsts, zrow)


def _run_sc_count(dsts, zcnt, ones_h):
    return _get_sc_count()(dsts, zcnt, ones_h)


def _proj_body(x_ref, w_ref, b_ref, lo_ref, hi_ref):
    h = lax.dot_general(x_ref[...], w_ref[...], (((1,), (1,)), ((), ())),
                        preferred_element_type=jnp.float32) + b_ref[...]
    lo_ref[...] = h[:, :HH]
    hi_ref[...] = h[:, HH:]


def _proj(x, w, b):
    return pl.pallas_call(
        _proj_body,
        grid=(GRID,),
        in_specs=[
            pl.BlockSpec((R, IN_DIM), lambda i: (i, 0)),
            pl.BlockSpec((HID, IN_DIM), lambda i: (0, 0)),
            pl.BlockSpec((1, HID), lambda i: (0, 0)),
        ],
        out_specs=[
            pl.BlockSpec((R, HH), lambda i: (i, 0)),
            pl.BlockSpec((R, HH), lambda i: (i, 0)),
        ],
        out_shape=[jax.ShapeDtypeStruct((N, HH), jnp.float32),
                   jax.ShapeDtypeStruct((N, HH), jnp.float32)],
    )(x, w, b.reshape(1, HID))


def _upd_body(s_lo_ref, s_hi_ref, cnt_ref, h_lo_ref, h_hi_ref,
              wl_ref, wr_ref, bias_ref, o_lo_ref, o_hi_ref):
    c = jnp.maximum(cnt_ref[:, 0:1], 1.0)
    mean = jnp.concatenate([s_lo_ref[0], s_hi_ref[0]], axis=1) / c
    h = jnp.concatenate([h_lo_ref[...], h_hi_ref[...]], axis=1)
    o = (lax.dot_general(mean, wl_ref[...], (((1,), (1,)), ((), ())),
                         preferred_element_type=jnp.float32)
         + lax.dot_general(h, wr_ref[...], (((1,), (1,)), ((), ())),
                           preferred_element_type=jnp.float32)
         + bias_ref[...])
    nh = jnp.maximum(o, 0.0) + h
    o_lo_ref[...] = nh[:, :HH]
    o_hi_ref[...] = nh[:, HH:]


def _upd(s2, cnt, h_lo, h_hi, wl, wr, bias):
    return pl.pallas_call(
        _upd_body,
        grid=(GRID,),
        in_specs=[
            pl.BlockSpec((1, R, HH), lambda i: (0, i, 0)),
            pl.BlockSpec((1, R, HH), lambda i: (1, i, 0)),
            pl.BlockSpec((R, HH), lambda i: (i, 0)),
            pl.BlockSpec((R, HH), lambda i: (i, 0)),
            pl.BlockSpec((R, HH), lambda i: (i, 0)),
            pl.BlockSpec((HID, HID), lambda i: (0, 0)),
            pl.BlockSpec((HID, HID), lambda i: (0, 0)),
            pl.BlockSpec((1, HID), lambda i: (0, 0)),
        ],
        out_specs=[
            pl.BlockSpec((R, HH), lambda i: (i, 0)),
            pl.BlockSpec((R, HH), lambda i: (i, 0)),
        ],
        out_shape=[jax.ShapeDtypeStruct((N, HH), jnp.float32),
                   jax.ShapeDtypeStruct((N, HH), jnp.float32)],
    )(s2, s2, cnt, h_lo, h_hi, wl, wr, bias)


def _pool_body(bu_ref, bi_ref, hu_lo_ref, hu_hi_ref, hi_lo_ref, hi_hi_ref,
               wout_ref, bout_ref, out_ref, su, si, cu, ci):
    i = pl.program_id(0)
    ones_r8 = jnp.ones((R, 8), jnp.float32)
    iota = lax.broadcasted_iota(jnp.int32, (NB, R), 0)

    oh_u = (iota == bu_ref[0]).astype(jnp.float32)
    oh_i = (iota == bi_ref[0]).astype(jnp.float32)
    hu = jnp.concatenate([hu_lo_ref[...], hu_hi_ref[...]], axis=1)
    hi = jnp.concatenate([hi_lo_ref[...], hi_hi_ref[...]], axis=1)
    dn = (((1,), (0,)), ((), ()))
    su_p = lax.dot_general(oh_u, hu, dn, preferred_element_type=jnp.float32)
    si_p = lax.dot_general(oh_i, hi, dn, preferred_element_type=jnp.float32)
    cu_p = lax.dot_general(oh_u, ones_r8, dn, preferred_element_type=jnp.float32)
    ci_p = lax.dot_general(oh_i, ones_r8, dn, preferred_element_type=jnp.float32)

    @pl.when(i == 0)
    def _():
        su[...] = su_p
        si[...] = si_p
        cu[...] = cu_p
        ci[...] = ci_p

    @pl.when(i > 0)
    def _():
        su[...] += su_p
        si[...] += si_p
        cu[...] += cu_p
        ci[...] += ci_p

    @pl.when(i == GRID - 1)
    def _():
        mu = su[...] / jnp.maximum(cu[...][:, 0:1], 1.0)
        mi = si[...] / jnp.maximum(ci[...][:, 0:1], 1.0)
        g = (mu + mi) * 0.5
        out_ref[...] = (lax.dot_general(g, wout_ref[...],
                                        (((1,), (1,)), ((), ())),
                                        preferred_element_type=jnp.float32)
                        + bout_ref[...])


def _pool(bu3, bi3, hu_lo, hu_hi, hi_lo, hi_hi, wout, bout):
    return pl.pallas_call(
        _pool_body,
        grid=(GRID,),
        in_specs=[
            pl.BlockSpec((1, 1, R), lambda i: (i, 0, 0)),
            pl.BlockSpec((1, 1, R), lambda i: (i, 0, 0)),
            pl.BlockSpec((R, HH), lambda i: (i, 0)),
            pl.BlockSpec((R, HH), lambda i: (i, 0)),
            pl.BlockSpec((R, HH), lambda i: (i, 0)),
            pl.BlockSpec((R, HH), lambda i: (i, 0)),
            pl.BlockSpec((HID, HID), lambda i: (0, 0)),
            pl.BlockSpec((1, HID), lambda i: (0, 0)),
        ],
        out_specs=pl.BlockSpec((NB, HID), lambda i: (0, 0)),
        out_shape=jax.ShapeDtypeStruct((NB, HID), jnp.float32),
        scratch_shapes=[
            pltpu.VMEM((NB, HID), jnp.float32),
            pltpu.VMEM((NB, HID), jnp.float32),
            pltpu.VMEM((NB, 8), jnp.float32),
            pltpu.VMEM((NB, 8), jnp.float32),
        ],
    )(bu3, bi3, hu_lo, hu_hi, hi_lo, hi_hi, wout, bout.reshape(1, HID))


def _prep_edges(ei):
    src = ei[0].astype(jnp.int32)
    dst = ei[1].astype(jnp.int32)
    pad = E_PAD - E
    src_p = jnp.concatenate([src, jnp.zeros((pad,), jnp.int32)])
    dst_p = jnp.concatenate([dst, jnp.full((pad,), N, jnp.int32)])
    srcs = jnp.stack([src_p, src_p + N]).reshape(NC, NS, CHUNKS, CH)
    dsts = dst_p.reshape(NS, CHUNKS, CH)
    return srcs, dsts


def kernel(x_user, x_item, edge_index_ui, edge_index_iu, batch_user, batch_item,
           Wp_user, bp_user, Wp_item, bp_item,
           Wl_ui_0, bl_ui_0, Wr_ui_0, br_ui_0,
           Wl_iu_0, bl_iu_0, Wr_iu_0, br_iu_0,
           Wl_ui_1, bl_ui_1, Wr_ui_1, br_ui_1,
           Wl_iu_1, bl_iu_1, Wr_iu_1, br_iu_1,
           Wout, bout):
    hu_lo, hu_hi = _proj(x_user, Wp_user, bp_user)
    hi_lo, hi_hi = _proj(x_item, Wp_item, bp_item)

    srcs_ui, dsts_ui = _prep_edges(edge_index_ui)
    srcs_iu, dsts_iu = _prep_edges(edge_index_iu)
    zrow = jnp.zeros((RPT, HH), jnp.float32)
    ones_h = jnp.ones((CH, HH), jnp.float32)
    c_i = _run_sc_count(dsts_ui, zrow, ones_h)
    c_u = _run_sc_count(dsts_iu, zrow, ones_h)

    layer_w = [(Wl_ui_0, Wr_ui_0, (bl_ui_0 + br_ui_0).reshape(1, HID),
                Wl_iu_0, Wr_iu_0, (bl_iu_0 + br_iu_0).reshape(1, HID)),
               (Wl_ui_1, Wr_ui_1, (bl_ui_1 + br_ui_1).reshape(1, HID),
                Wl_iu_1, Wr_iu_1, (bl_iu_1 + br_iu_1).reshape(1, HID))]

    for (wl_ui, wr_ui, b_ui, wl_iu, wr_iu, b_iu) in layer_w:
        table_u = jnp.concatenate([hu_lo, hu_hi], axis=0)
        table_i = jnp.concatenate([hi_lo, hi_hi], axis=0)
        s_i = _run_sc_round(table_u, srcs_ui, dsts_ui, zrow)
        s_u = _run_sc_round(table_i, srcs_iu, dsts_iu, zrow)
        hi_lo, hi_hi = _upd(s_i, c_i[:N], hi_lo, hi_hi, wl_ui, wr_ui, b_ui)
        hu_lo, hu_hi = _upd(s_u, c_u[:N], hu_lo, hu_hi, wl_iu, wr_iu, b_iu)

    bu3 = batch_user.astype(jnp.int32).reshape(GRID, 1, R)
    bi3 = batch_item.astype(jnp.int32).reshape(GRID, 1, R)
    return _pool(bu3, bi3, hu_lo, hu_hi, hi_lo, hi_hi, Wout, bout)
